# Initial kernel scaffold; baseline (speedup 1.0000x reference)
import jax
import jax.numpy as jnp
from jax.experimental import pallas as pl


def kernel(pred_logits, gt, mask):
    raise NotImplementedError("write your pallas kernel here")


# speedup vs baseline: 64.1861x; 64.1861x over previous
import functools

import jax
import jax.numpy as jnp
from jax.experimental import pallas as pl
from jax.experimental.pallas import tpu as pltpu

_ROWS = 2048
_COLS = 1024
_BLK = 256
_NBLK = _ROWS // _BLK
_EPS = 1e-6


def _body(x_ref, z_ref, m_ref, out_ref, bits_ref, acc_ref):
    i = pl.program_id(0)

    @pl.when(i == 0)
    def _init():
        acc_ref[0] = 0.0
        acc_ref[1] = 0.0
        acc_ref[2] = 0.0
        acc_ref[3] = 0.0

    x = x_ref[...]
    z = z_ref[...]
    m = m_ref[...]
    loss = jnp.maximum(x, 0.0) - x * z + jnp.log1p(jnp.exp(-jnp.abs(x)))
    posf = ((z * m) > 0.0).astype(jnp.float32)
    negf = (((1.0 - z) * m) > 0.0).astype(jnp.float32)
    acc_ref[0] += jnp.sum(posf)
    acc_ref[1] += jnp.sum(negf)
    acc_ref[2] += jnp.sum(loss * posf)
    acc_ref[3] += jnp.sum(loss * negf)
    bits_ref[pl.ds(i * _BLK, _BLK), :] = jax.lax.bitcast_convert_type(
        loss * negf, jnp.int32
    )

    @pl.when(i == _NBLK - 1)
    def _finish():
        pos_cnt = acc_ref[0]
        neg_cnt = acc_ref[1]
        pos_sum = acc_ref[2]
        neg_sum = acc_ref[3]
        k = jnp.minimum(neg_cnt, jnp.floor(pos_cnt * 3.0))
        out_ref[0] = (pos_sum + neg_sum) / (pos_cnt + neg_cnt + _EPS)

        @pl.when(k < neg_cnt)
        def _select():
            def _count_ge(mid):
                def cbody(c, a):
                    blk = bits_ref[pl.ds(c * _BLK, _BLK), :]
                    return a + jnp.sum((blk >= mid).astype(jnp.float32))

                return jax.lax.fori_loop(0, _NBLK, cbody, 0.0)

            def bbody(_, lohi):
                lo, hi = lohi
                mid = lo + jax.lax.div(hi - lo, 2)
                pred = _count_ge(mid) >= k
                return (jnp.where(pred, mid, lo), jnp.where(pred, hi, mid))

            lo, _hi = jax.lax.fori_loop(
                0, 31, bbody, (jnp.int32(0), jnp.int32(0x7F800000))
            )
            t = jax.lax.bitcast_convert_type(lo, jnp.float32)

            def sbody(c, carry):
                cnt, s = carry
                blk = bits_ref[pl.ds(c * _BLK, _BLK), :]
                sel = blk > lo
                vals = jax.lax.bitcast_convert_type(blk, jnp.float32)
                cnt += jnp.sum(sel.astype(jnp.float32))
                s += jnp.sum(jnp.where(sel, vals, 0.0))
                return (cnt, s)

            cnt_gt, sum_gt = jax.lax.fori_loop(0, _NBLK, sbody, (0.0, 0.0))
            topk = sum_gt + (k - cnt_gt) * t
            out_ref[0] = (pos_sum + topk) / (pos_cnt + k + _EPS)


@functools.partial(jax.jit, static_argnames=("interpret",))
def kernel(pred_logits, gt, mask, interpret=False):
    x = pred_logits.reshape(_ROWS, _COLS)
    z = gt.reshape(_ROWS, _COLS)
    m = mask.reshape(_ROWS, _COLS)
    spec = pl.BlockSpec((_BLK, _COLS), lambda i: (i, 0))
    out = pl.pallas_call(
        _body,
        grid=(_NBLK,),
        in_specs=[spec, spec, spec],
        out_specs=pl.BlockSpec(memory_space=pltpu.SMEM),
        out_shape=jax.ShapeDtypeStruct((1,), jnp.float32),
        scratch_shapes=[
            pltpu.VMEM((_ROWS, _COLS), jnp.int32),
            pltpu.SMEM((4,), jnp.float32),
        ],
        interpret=interpret,
    )(x, z, m)
    return out[0]
